# baseline (speedup 1.0000x reference)
import jax
import jax.numpy as jnp
from jax.experimental import pallas as pl
from jax.experimental.pallas import tpu as pltpu
from jax.experimental.pallas import tpu_sc as plsc

EMBED_DIM = 100
PAD_DIM = 128
B_BLK = 4
WAVE = 2
LANES = 16
OFFS = (0, 16, 32, 48, 64, 80, EMBED_DIM - LANES)
DIR_REP = 128


def _tc_pad_table(table):
    v = table.shape[0]
    blk = 1000 if v % 1000 == 0 else v

    def body(t_ref, o_ref):
        o_ref[...] = jnp.concatenate(
            [t_ref[...], jnp.zeros((blk, PAD_DIM - EMBED_DIM), jnp.float32)],
            axis=1,
        )

    return pl.pallas_call(
        body,
        grid=(v // blk,),
        in_specs=[pl.BlockSpec((blk, EMBED_DIM), lambda i: (i, 0))],
        out_specs=pl.BlockSpec((blk, PAD_DIM), lambda i: (i, 0)),
        out_shape=jax.ShapeDtypeStruct((v, PAD_DIM), jnp.float32),
    )(table)


def _sc_all(iat_pad, pkt_pad, dir_tab, iat_seq, pkt_seq, dir_idx, sid_arr,
            batch, seq):
    mesh = plsc.VectorSubcoreMesh(core_axis_name="c", subcore_axis_name="s")
    out_struct = jax.ShapeDtypeStruct((batch, seq, EMBED_DIM), jnp.float32)

    @pl.kernel(
        out_type=(out_struct, out_struct, out_struct),
        mesh=mesh,
        scratch_types=[
            pltpu.VMEM((B_BLK, seq, PAD_DIM), jnp.float32),
            pltpu.VMEM((B_BLK, seq, PAD_DIM), jnp.float32),
            pltpu.VMEM((WAVE, seq, PAD_DIM), jnp.float32),
            pltpu.VMEM((WAVE, seq, EMBED_DIM), jnp.float32),
            pltpu.VMEM((WAVE, seq, EMBED_DIM), jnp.float32),
            pltpu.VMEM((WAVE, seq, EMBED_DIM), jnp.float32),
            pltpu.SemaphoreType.DMA,
            pltpu.SemaphoreType.DMA,
            pltpu.SemaphoreType.DMA,
        ],
    )
    def k(iat_t, pkt_t, dir_t, ii_h, pi_h, di_h, sid_h, io_h, po_h, do_h,
          g_i, g_p, g_d, n_i_s, n_p_s, n_d_s,
          gsemA, gsemB, wsem):
        outs = (io_h, po_h, do_h)

        def wave_writes(bufs, sem, b0, j0):
            return [
                pltpu.make_async_copy(buf.at[jj], out.at[b0 + j0 + jj], sem)
                for buf, out in zip(bufs, outs)
                for jj in range(WAVE)
            ]

        def prime_writes(bufs, sem):
            for buf, out in zip(bufs, outs):
                for jj in range(WAVE):
                    pltpu.make_async_copy(out.at[0], buf.at[jj], sem).start()

        def narrow(bufs, j0):
            n_i, n_p, n_d = bufs

            @pl.loop(0, seq)
            def _(r):
                for jj in range(WAVE):
                    j = j0 + jj
                    for off in OFFS:
                        sl = pl.ds(off, LANES)
                        n_i[jj, r, sl] = g_i[j, r, sl]
                        n_p[jj, r, sl] = g_p[j, r, sl]
                        n_d[jj, r, sl] = g_d[jj, r, sl]

        nn = (n_i_s, n_p_s, n_d_s)

        prime_writes(nn, wsem)

        def body(ii_vmem, pi_vmem, di_vmem, sid_vmem):
            sid_row = sid_vmem.at[0][...]
            sid = jax.lax.squeeze(jax.lax.slice(sid_row, (0,), (1,)), (0,))
            b0 = sid * B_BLK

            ga, gb = [], []
            for j in range(B_BLK):
                sem = gsemA if j < WAVE else gsemB
                dst = ga if j < WAVE else gb
                dst.append(pltpu.async_copy(
                    iat_t.at[ii_vmem.at[j]], g_i.at[j], sem))
                dst.append(pltpu.async_copy(
                    pkt_t.at[pi_vmem.at[j]], g_p.at[j], sem))
            for jj in range(WAVE):
                ga.append(pltpu.async_copy(
                    dir_t.at[di_vmem.at[jj]], g_d.at[jj], gsemA))

            for g in ga:
                g.wait()
            for h in wave_writes(nn, wsem, 0, 0):
                h.wait()
            narrow(nn, 0)
            for h in wave_writes(nn, wsem, b0, 0):
                h.start()

            gd2 = [pltpu.async_copy(
                dir_t.at[di_vmem.at[WAVE + jj]], g_d.at[jj], gsemB)
                for jj in range(WAVE)]
            for g in gb + gd2:
                g.wait()
            for h in wave_writes(nn, wsem, 0, 0):
                h.wait()
            narrow(nn, WAVE)
            for h in wave_writes(nn, wsem, b0, WAVE):
                h.start()

        pltpu.emit_pipeline(
            body,
            grid=(batch // B_BLK,),
            in_specs=[
                pl.BlockSpec((B_BLK, seq), lambda i: (i, 0)),
                pl.BlockSpec((B_BLK, seq), lambda i: (i, 0)),
                pl.BlockSpec((B_BLK, seq), lambda i: (i, 0)),
                pl.BlockSpec((1, 16), lambda i: (i, 0)),
            ],
            core_axis_name=("c", "s"),
            dimension_semantics=(pltpu.PARALLEL,),
        )(ii_h, pi_h, di_h, sid_h)

        for h in wave_writes(nn, wsem, 0, 0):
            h.wait()

    return k(iat_pad, pkt_pad, dir_tab, iat_seq, pkt_seq, dir_idx, sid_arr)


def kernel(pkt_len_seq, pkt_dir_seq, iat_seq, pkt_len_table, iat_table):
    batch, seq = pkt_len_seq.shape

    iat_pad = jnp.pad(iat_table, ((0, 0), (0, PAD_DIM - EMBED_DIM)))
    pkt_pad = jnp.pad(pkt_len_table, ((0, 0), (0, PAD_DIM - EMBED_DIM)))
    dir_tab = jnp.concatenate([
        jnp.full((DIR_REP, PAD_DIM), -1.0, jnp.float32),
        jnp.full((DIR_REP, PAD_DIM), 1.0, jnp.float32),
    ])
    stripe = jnp.broadcast_to(
        jnp.arange(seq, dtype=jnp.int32)[None, :] % DIR_REP, (batch, seq))
    dir_bit = (pkt_dir_seq.astype(jnp.int32) + 1) >> 1
    dir_idx = dir_bit * DIR_REP + stripe
    sid_arr = jnp.broadcast_to(
        jnp.arange(batch // B_BLK, dtype=jnp.int32)[:, None],
        (batch // B_BLK, 16))

    iat_out, pkt_out, dir_out = _sc_all(
        iat_pad, pkt_pad, dir_tab,
        iat_seq.astype(jnp.int32), pkt_len_seq.astype(jnp.int32), dir_idx,
        sid_arr, batch, seq,
    )
    return (pkt_out, dir_out, iat_out)


# speedup vs baseline: 1.7025x; 1.7025x over previous
import jax
import jax.numpy as jnp
from jax.experimental import pallas as pl
from jax.experimental.pallas import tpu as pltpu
from jax.experimental.pallas import tpu_sc as plsc

EMBED_DIM = 100
PAD_DIM = 128
B_BLK = 4
LANES = 16
OFFS = (0, 16, 32, 48, 64, 80, EMBED_DIM - LANES)


def _tc_pad_table(table):
    v = table.shape[0]
    blk = 1000 if v % 1000 == 0 else v

    def body(t_ref, o_ref):
        o_ref[...] = jnp.concatenate(
            [t_ref[...], jnp.zeros((blk, PAD_DIM - EMBED_DIM), jnp.float32)],
            axis=1,
        )

    return pl.pallas_call(
        body,
        grid=(v // blk,),
        in_specs=[pl.BlockSpec((blk, EMBED_DIM), lambda i: (i, 0))],
        out_specs=pl.BlockSpec((blk, PAD_DIM), lambda i: (i, 0)),
        out_shape=jax.ShapeDtypeStruct((v, PAD_DIM), jnp.float32),
    )(table)


def _sc_gather(iat_pad, pkt_pad, iat_seq, pkt_len_seq, step_ids, batch, seq):
    mesh = plsc.VectorSubcoreMesh(core_axis_name="c", subcore_axis_name="s")
    out_struct = jax.ShapeDtypeStruct((batch, seq, EMBED_DIM), jnp.float32)

    @pl.kernel(
        out_type=(out_struct, out_struct),
        mesh=mesh,
        scratch_types=[
            pltpu.VMEM((B_BLK, seq, PAD_DIM), jnp.float32),
            pltpu.VMEM((B_BLK, seq, PAD_DIM), jnp.float32),
            pltpu.VMEM((B_BLK, seq, EMBED_DIM), jnp.float32),
            pltpu.VMEM((B_BLK, seq, EMBED_DIM), jnp.float32),
            pltpu.SemaphoreType.DMA,
            pltpu.SemaphoreType.DMA,
        ],
    )
    def k(iat_t_hbm, pkt_t_hbm, iat_i_hbm, pkt_i_hbm, sid_hbm,
          iat_o_hbm, pkt_o_hbm, ig_v, pg_v, in_v, pn_v, gsem, wsem):
        def body(ii_vmem, pi_vmem, sid_vmem):
            sid_row = sid_vmem.at[0][...]
            b0 = jax.lax.squeeze(jax.lax.slice(sid_row, (0,), (1,)), (0,)) * B_BLK
            gathers = []
            for j in range(B_BLK):
                gathers.append(
                    pltpu.async_copy(iat_t_hbm.at[ii_vmem.at[j]], ig_v.at[j], gsem))
                gathers.append(
                    pltpu.async_copy(pkt_t_hbm.at[pi_vmem.at[j]], pg_v.at[j], gsem))
            for g in gathers:
                g.wait()

            @pl.loop(0, seq)
            def _(r):
                for j in range(B_BLK):
                    for off in OFFS:
                        sl = pl.ds(off, LANES)
                        in_v[j, r, sl] = ig_v[j, r, sl]
                        pn_v[j, r, sl] = pg_v[j, r, sl]

            writes = []
            for j in range(B_BLK):
                writes.append(pltpu.async_copy(
                    in_v.at[j], iat_o_hbm.at[b0 + j], wsem))
                writes.append(pltpu.async_copy(
                    pn_v.at[j], pkt_o_hbm.at[b0 + j], wsem))
            for w in writes:
                w.wait()

        pltpu.emit_pipeline(
            body,
            grid=(batch // B_BLK,),
            in_specs=[
                pl.BlockSpec((B_BLK, seq), lambda i: (i, 0)),
                pl.BlockSpec((B_BLK, seq), lambda i: (i, 0)),
                pl.BlockSpec((1, 16), lambda i: (i, 0)),
            ],
            core_axis_name=("c", "s"),
            dimension_semantics=(pltpu.PARALLEL,),
        )(iat_i_hbm, pkt_i_hbm, sid_hbm)

    return k(iat_pad, pkt_pad, iat_seq, pkt_len_seq, step_ids)


def _tc_dir_broadcast(pkt_dir_seq, batch, seq):
    b_blk = 256

    def body(d_ref, o_ref):
        o_ref[...] = jnp.broadcast_to(
            d_ref[...].astype(jnp.float32)[:, :, None], (b_blk, seq, EMBED_DIM)
        )

    return pl.pallas_call(
        body,
        grid=(batch // b_blk,),
        in_specs=[pl.BlockSpec((b_blk, seq), lambda i: (i, 0))],
        out_specs=pl.BlockSpec((b_blk, seq, EMBED_DIM), lambda i: (i, 0, 0)),
        out_shape=jax.ShapeDtypeStruct((batch, seq, EMBED_DIM), jnp.float32),
    )(pkt_dir_seq)


def kernel(pkt_len_seq, pkt_dir_seq, iat_seq, pkt_len_table, iat_table):
    batch, seq = pkt_len_seq.shape

    iat_pad = _tc_pad_table(iat_table)
    pkt_pad = _tc_pad_table(pkt_len_table)
    step_ids = jnp.broadcast_to(
        jnp.arange(batch // B_BLK, dtype=jnp.int32)[:, None], (batch // B_BLK, 16)
    )

    iat_out, pkt_out = _sc_gather(
        iat_pad, pkt_pad,
        iat_seq.astype(jnp.int32), pkt_len_seq.astype(jnp.int32),
        step_ids, batch, seq,
    )
    dir_out = _tc_dir_broadcast(pkt_dir_seq, batch, seq)

    return (pkt_out, dir_out, iat_out)
